# baseline (speedup 1.0000x reference)
import functools

import jax
import jax.numpy as jnp
from jax import lax
from jax.experimental import pallas as pl
from jax.experimental.pallas import tpu as pltpu
from jax.experimental.pallas import tpu_sc as plsc

_BATCH = 16384
_FIELDS = 26
_DIM = 64
_VOCAB = 1000000
_HALF = _VOCAB // 2
_NC = 2
_NS = 16
_NW = _NC * _NS
_EPW = _BATCH // _NW
_CB = 128
_NCH = _EPW // _CB
_NIT = _FIELDS * _NCH

_mesh = plsc.VectorSubcoreMesh(
    core_axis_name="c", subcore_axis_name="s", num_cores=_NC, num_subcores=_NS
)

_DNUMS = lax.GatherDimensionNumbers(
    offset_dims=(), collapsed_slice_dims=(0,), start_index_map=(0,)
)


def _lane_perm(v, p):
    return lax.gather(
        v, p[:, None], _DNUMS, (1,), mode=lax.GatherScatterMode.PROMISE_IN_BOUNDS
    )


@functools.partial(
    pl.kernel,
    out_type=jax.ShapeDtypeStruct(
        (_FIELDS, _DIM // 8, _BATCH // 128, 8, 128), jnp.float32
    ),
    mesh=_mesh,
    scratch_types=(
        [
            pltpu.VMEM((_FIELDS, _EPW), jnp.int32),
            pltpu.VMEM((_FIELDS, _EPW), jnp.int32),
            pltpu.VMEM((_FIELDS, _EPW), jnp.int32),
            pltpu.VMEM((_CB, _DIM), jnp.float32),
            pltpu.VMEM((_CB, _DIM), jnp.float32),
            pltpu.VMEM((_CB, _DIM), jnp.float32),
            pltpu.VMEM((_CB, _DIM), jnp.float32),
            pltpu.VMEM((8, 8, _CB), jnp.float32),
            pltpu.VMEM((8, 8, _CB), jnp.float32),
        ]
        + [pltpu.SemaphoreType.DMA] * 4
    ),
    compiler_params=pltpu.CompilerParams(use_tc_tiling_on_sc=False),
)
def _gather_kernel(
    idx_hbm, tlo_hbm, thi_hbm, out_hbm,
    idx_v, lo_v, hi_v, rl0, rl1, rh0, rh1, s0, s1, *sems,
):
    rows_lo = (rl0, rl1)
    rows_hi = (rh0, rh1)
    stage = (s0, s1)
    gsem = sems[:2]
    osem = sems[2:]
    wid = lax.axis_index("s") * _NC + lax.axis_index("c")
    b0 = wid * _EPW
    pltpu.sync_copy(idx_hbm.at[:, pl.ds(b0, _EPW)], idx_v)

    lane = lax.iota(jnp.int32, 16)
    perms = {k: lane ^ k for k in (1, 2, 4, 8)}
    masks = {
        (k, ik): (lane & k) == ik for k in (1, 2, 4, 8) for ik in (0, k)
    }

    def clamp_body(f, carry):
        for g in range(_EPW // 16):
            v = idx_v[f, pl.ds(16 * g, 16)]
            lo_v[f, pl.ds(16 * g, 16)] = jnp.minimum(v, _HALF - 1)
            hi_v[f, pl.ds(16 * g, 16)] = jnp.maximum(v - _HALF, 0)
        return carry

    lax.fori_loop(0, _FIELDS, clamp_body, 0)

    def g_copies(i, b):
        f = i // _NCH
        c = lax.rem(i, _NCH)
        lo = pltpu.make_async_copy(
            tlo_hbm.at[lo_v.at[f, pl.ds(c * _CB, _CB)]], rows_lo[b], gsem[b]
        )
        hi = pltpu.make_async_copy(
            thi_hbm.at[hi_v.at[f, pl.ds(c * _CB, _CB)]], rows_hi[b], gsem[b]
        )
        return lo, hi

    def o_copy(i, b):
        f = i // _NCH
        c = lax.rem(i, _NCH)
        ct = b0 // 128 + c
        return pltpu.make_async_copy(
            stage[b], out_hbm.at[f, :, ct, :, :], osem[b]
        )

    def butterfly(r):
        for k in (1, 2, 4, 8):
            r = [
                jnp.where(masks[(k, i & k)], r[i], _lane_perm(r[i ^ k], perms[k]))
                for i in range(16)
            ]
        return r

    def transpose(i, b):
        f = i // _NCH
        c = lax.rem(i, _NCH)
        src_lo = rows_lo[b]
        src_hi = rows_hi[b]
        dst = stage[b]

        def t_body(bb, carry):
            bl = bb * 16
            sel = idx_v[f, pl.ds(c * _CB + bl, 16)] >= _HALF
            for db in range(_DIM // 16):
                d0 = db * 16
                rl = butterfly([src_lo[bl + l, pl.ds(d0, 16)] for l in range(16)])
                rh = butterfly([src_hi[bl + l, pl.ds(d0, 16)] for l in range(16)])
                for j in range(16):
                    d = d0 + j
                    dst[d // 8, d % 8, pl.ds(bl, 16)] = jnp.where(sel, rh[j], rl[j])
            return carry

        lax.fori_loop(0, _CB // 16, t_body, 0)

    for cp in g_copies(0, 0):
        cp.start()
    for cp in g_copies(1, 1):
        cp.start()

    def body(p, carry):
        for b in range(2):
            i = 2 * p + b
            for cp in g_copies(i, b):
                cp.wait()

            @pl.when(i >= 2)
            def _():
                o_copy(i - 2, b).wait()

            transpose(i, b)
            o_copy(i, b).start()

            @pl.when(i + 2 < _NIT)
            def _():
                for cp in g_copies(i + 2, b):
                    cp.start()

        return carry

    lax.fori_loop(0, _NIT // 2, body, 0)

    o_copy(_NIT - 2, 0).wait()
    o_copy(_NIT - 1, 1).wait()


def kernel(indices, table):
    idx_t = indices.T
    t_lo = lax.slice_in_dim(table, 0, _HALF, axis=0)
    t_hi = lax.slice_in_dim(table, _HALF, _VOCAB, axis=0)
    out5 = _gather_kernel(idx_t, t_lo, t_hi)
    return jnp.transpose(out5, (2, 4, 0, 1, 3)).reshape(_BATCH, _FIELDS, _DIM)


# speedup vs baseline: 7.0409x; 7.0409x over previous
import functools

import jax
import jax.numpy as jnp
from jax import lax
from jax.experimental import pallas as pl
from jax.experimental.pallas import tpu as pltpu
from jax.experimental.pallas import tpu_sc as plsc

_BATCH = 16384
_FIELDS = 26
_DIM = 64
_NC = 2
_NS = 16
_NW = _NC * _NS
_EPW = _BATCH // _NW
_CB = 128
_NCH = _EPW // _CB
_NIT = _FIELDS * _NCH

_mesh = plsc.VectorSubcoreMesh(
    core_axis_name="c", subcore_axis_name="s", num_cores=_NC, num_subcores=_NS
)

_DNUMS = lax.GatherDimensionNumbers(
    offset_dims=(), collapsed_slice_dims=(0,), start_index_map=(0,)
)


def _lane_perm(v, p):
    return lax.gather(
        v, p[:, None], _DNUMS, (1,), mode=lax.GatherScatterMode.PROMISE_IN_BOUNDS
    )


@functools.partial(
    pl.kernel,
    out_type=jax.ShapeDtypeStruct(
        (_FIELDS, _DIM // 8, _BATCH // 128, 8, 128), jnp.float32
    ),
    mesh=_mesh,
    scratch_types=(
        [
            pltpu.VMEM((_FIELDS, _EPW), jnp.int32),
            pltpu.VMEM((_CB, _DIM), jnp.float32),
            pltpu.VMEM((_CB, _DIM), jnp.float32),
            pltpu.VMEM((8, 8, _CB), jnp.float32),
            pltpu.VMEM((8, 8, _CB), jnp.float32),
        ]
        + [pltpu.SemaphoreType.DMA] * 4
    ),
    compiler_params=pltpu.CompilerParams(use_tc_tiling_on_sc=False),
)
def _gather_kernel(idx_hbm, table_hbm, out_hbm, idx_v, r0, r1, s0, s1, *sems):
    rows = (r0, r1)
    stage = (s0, s1)
    gsem = sems[:2]
    osem = sems[2:]
    wid = lax.axis_index("s") * _NC + lax.axis_index("c")
    b0 = wid * _EPW
    pltpu.sync_copy(idx_hbm.at[:, pl.ds(b0, _EPW)], idx_v)

    lane = lax.iota(jnp.int32, 16)
    perms = {k: lane ^ k for k in (1, 2, 4, 8)}
    masks = {
        (k, ik): (lane & k) == ik for k in (1, 2, 4, 8) for ik in (0, k)
    }

    def g_copy(i, b):
        f = i // _NCH
        c = lax.rem(i, _NCH)
        return pltpu.make_async_copy(
            table_hbm.at[idx_v.at[f, pl.ds(c * _CB, _CB)]], rows[b], gsem[b]
        )

    def o_copy(i, b):
        f = i // _NCH
        c = lax.rem(i, _NCH)
        ct = b0 // 128 + c
        return pltpu.make_async_copy(
            stage[b], out_hbm.at[f, :, ct, :, :], osem[b]
        )

    def transpose(b):
        src = rows[b]
        dst = stage[b]

        def t_body(bb, carry):
            bl = bb * 16
            for db in range(_DIM // 16):
                d0 = db * 16
                r = [src[bl + l, pl.ds(d0, 16)] for l in range(16)]
                for k in (1, 2, 4, 8):
                    r = [
                        jnp.where(
                            masks[(k, i & k)], r[i], _lane_perm(r[i ^ k], perms[k])
                        )
                        for i in range(16)
                    ]
                for j in range(16):
                    d = d0 + j
                    dst[d // 8, d % 8, pl.ds(bl, 16)] = r[j]
            return carry

        lax.fori_loop(0, _CB // 16, t_body, 0)

    g_copy(0, 0).start()
    g_copy(1, 1).start()

    def body(p, carry):
        for b in range(2):
            i = 2 * p + b
            g_copy(i, b).wait()

            @pl.when(i >= 2)
            def _():
                o_copy(i - 2, b).wait()

            transpose(b)
            o_copy(i, b).start()

            @pl.when(i + 2 < _NIT)
            def _():
                g_copy(i + 2, b).start()

        return carry

    lax.fori_loop(0, _NIT // 2, body, 0)

    o_copy(_NIT - 2, 0).wait()
    o_copy(_NIT - 1, 1).wait()


def kernel(indices, table):
    idx_t = indices.T
    out5 = _gather_kernel(idx_t, table)
    return jnp.transpose(out5, (2, 4, 0, 1, 3)).reshape(_BATCH, _FIELDS, _DIM)


# speedup vs baseline: 7.0632x; 1.0032x over previous
import functools

import jax
import jax.numpy as jnp
from jax import lax
from jax.experimental import pallas as pl
from jax.experimental.pallas import tpu as pltpu
from jax.experimental.pallas import tpu_sc as plsc

_BATCH = 16384
_FIELDS = 26
_DIM = 64
_NC = 2
_NS = 16
_NW = _NC * _NS
_EPW = _BATCH // _NW
_CB = 128
_NCH = _EPW // _CB
_NIT = _FIELDS * _NCH

_mesh = plsc.VectorSubcoreMesh(
    core_axis_name="c", subcore_axis_name="s", num_cores=_NC, num_subcores=_NS
)

_DNUMS = lax.GatherDimensionNumbers(
    offset_dims=(), collapsed_slice_dims=(0,), start_index_map=(0,)
)


def _lane_perm(v, p):
    return lax.gather(
        v, p[:, None], _DNUMS, (1,), mode=lax.GatherScatterMode.PROMISE_IN_BOUNDS
    )


@functools.partial(
    pl.kernel,
    out_type=jax.ShapeDtypeStruct(
        (_FIELDS, _DIM // 8, _BATCH // 128, 8, 128), jnp.float32
    ),
    mesh=_mesh,
    scratch_types=(
        [
            pltpu.VMEM((_FIELDS, _EPW), jnp.int32),
            pltpu.VMEM((_CB, _DIM), jnp.float32),
            pltpu.VMEM((_CB, _DIM), jnp.float32),
            pltpu.VMEM((_CB, _DIM), jnp.float32),
            pltpu.VMEM((_CB, _DIM), jnp.float32),
            pltpu.VMEM((8, 8, _CB), jnp.float32),
            pltpu.VMEM((8, 8, _CB), jnp.float32),
            pltpu.VMEM((8, 8, _CB), jnp.float32),
            pltpu.VMEM((8, 8, _CB), jnp.float32),
        ]
        + [pltpu.SemaphoreType.DMA] * 8
    ),
    compiler_params=pltpu.CompilerParams(use_tc_tiling_on_sc=False),
)
def _gather_kernel(
    idx_hbm, table_hbm, out_hbm, idx_v, r0, r1, r2, r3, s0, s1, s2, s3, *sems
):
    rows = (r0, r1, r2, r3)
    stage = (s0, s1, s2, s3)
    gsem = sems[:4]
    osem = sems[4:]
    wid = lax.axis_index("s") * _NC + lax.axis_index("c")
    b0 = wid * _EPW
    pltpu.sync_copy(idx_hbm.at[:, pl.ds(b0, _EPW)], idx_v)

    lane = lax.iota(jnp.int32, 16)
    perms = {k: lane ^ k for k in (1, 2, 4, 8)}
    masks = {
        (k, ik): (lane & k) == ik for k in (1, 2, 4, 8) for ik in (0, k)
    }

    def g_copy(i, b):
        f = i // _NCH
        c = lax.rem(i, _NCH)
        return pltpu.make_async_copy(
            table_hbm.at[idx_v.at[f, pl.ds(c * _CB, _CB)]], rows[b], gsem[b]
        )

    def o_copy(i, b):
        f = i // _NCH
        c = lax.rem(i, _NCH)
        ct = b0 // 128 + c
        return pltpu.make_async_copy(
            stage[b], out_hbm.at[f, :, ct, :, :], osem[b]
        )

    def transpose(b):
        src = rows[b]
        dst = stage[b]

        def t_body(bb, carry):
            bl = bb * 16
            for db in range(_DIM // 16):
                d0 = db * 16
                r = [src[bl + l, pl.ds(d0, 16)] for l in range(16)]
                for k in (1, 2, 4, 8):
                    r = [
                        jnp.where(
                            masks[(k, i & k)], r[i], _lane_perm(r[i ^ k], perms[k])
                        )
                        for i in range(16)
                    ]
                for j in range(16):
                    d = d0 + j
                    dst[d // 8, d % 8, pl.ds(bl, 16)] = r[j]
            return carry

        lax.fori_loop(0, _CB // 16, t_body, 0)

    for b in range(4):
        g_copy(b, b).start()

    def body(p, carry):
        for b in range(4):
            i = 4 * p + b
            g_copy(i, b).wait()

            @pl.when(i >= 4)
            def _():
                o_copy(i - 4, b).wait()

            transpose(b)
            o_copy(i, b).start()

            @pl.when(i + 4 < _NIT)
            def _():
                g_copy(i + 4, b).start()

        return carry

    lax.fori_loop(0, _NIT // 4, body, 0)

    for b in range(4):
        o_copy(_NIT - 4 + b, b).wait()


def kernel(indices, table):
    idx_t = indices.T
    out5 = _gather_kernel(idx_t, table)
    return jnp.transpose(out5, (2, 4, 0, 1, 3)).reshape(_BATCH, _FIELDS, _DIM)
